# baseline (speedup 1.0000x reference)
import functools

import jax
import jax.numpy as jnp
from jax import lax
from jax.experimental import pallas as pl
from jax.experimental.pallas import tpu as pltpu
from jax.experimental.pallas import tpu_sc as plsc

TEMP_INV = 10.0

B = 16384
C0 = 64
C1 = 256
D = 32

NC, NS = 2, 16
NW = NC * NS
BPW = B // NW

A1_CHUNK = 64
A1_NCHUNK = BPW // A1_CHUNK
ROW_K = 16
ROW_NCHUNK = BPW // ROW_K


def _sc_gather(uid, iid, a0, a1, it):
    mesh = plsc.VectorSubcoreMesh(core_axis_name="c", subcore_axis_name="s")

    @functools.partial(
        pl.kernel,
        mesh=mesh,
        out_type=(
            jax.ShapeDtypeStruct((B, C0), jnp.float32),
            jax.ShapeDtypeStruct((B, C1), jnp.float32),
            jax.ShapeDtypeStruct((B, D), jnp.float32),
        ),
        scratch_types=[
            pltpu.VMEM((BPW,), jnp.int32),
            pltpu.VMEM((BPW,), jnp.int32),
            pltpu.VMEM((A1_NCHUNK, A1_CHUNK), jnp.int32),
            pltpu.VMEM((A1_CHUNK, C1), jnp.float32),
            pltpu.VMEM((A1_CHUNK, C1), jnp.float32),
            pltpu.SemaphoreType.DMA,
            pltpu.SemaphoreType.DMA,
            pltpu.SemaphoreType.DMA,
            pltpu.SemaphoreType.DMA,
        ],
    )
    def k(uid_hbm, iid_hbm, a0_hbm, a1_hbm, it_hbm,
          g0_hbm, g1_hbm, v_hbm,
          uid_v, iid_v, uidx1_v, a1_p, a1_q,
          s1p, s1q, s0, si):
        wid = lax.axis_index("s") * NC + lax.axis_index("c")
        base = wid * BPW
        pltpu.sync_copy(uid_hbm.at[pl.ds(base, BPW)], uid_v)
        pltpu.sync_copy(iid_hbm.at[pl.ds(base, BPW)], iid_v)
        for j in range(A1_NCHUNK):
            pltpu.sync_copy(
                uid_hbm.at[pl.ds(base + j * A1_CHUNK, A1_CHUNK)],
                uidx1_v.at[j])

        def fire1(kk):
            return pltpu.async_copy(
                a1_hbm.at[uidx1_v.at[kk]],
                (a1_p, a1_q)[kk % 2], (s1p, s1q)[kk % 2])

        cp1 = [None] * A1_NCHUNK
        cp1[0] = fire1(0)
        cp1[1] = fire1(1)

        rows_per_a1 = ROW_NCHUNK // A1_NCHUNK
        for rnd in range(A1_NCHUNK):
            @pl.loop(rnd * rows_per_a1, (rnd + 1) * rows_per_a1)
            def _row_group(cg):
                uvec = uid_v[pl.ds(cg * ROW_K, ROW_K)]
                ivec = iid_v[pl.ds(cg * ROW_K, ROW_K)]
                cps = []
                for t in range(ROW_K):
                    j = cg * ROW_K + t
                    u = uvec[t]
                    i = ivec[t]
                    cps.append(pltpu.async_copy(
                        a0_hbm.at[u], g0_hbm.at[base + j], s0))
                    cps.append(pltpu.async_copy(
                        it_hbm.at[i], v_hbm.at[base + j], si))
                for cp in cps:
                    cp.wait()
            cp1[rnd].wait()
            pltpu.sync_copy(
                (a1_p, a1_q)[rnd % 2],
                g1_hbm.at[pl.ds(base + rnd * A1_CHUNK, A1_CHUNK)])
            if rnd + 2 < A1_NCHUNK:
                cp1[rnd + 2] = fire1(rnd + 2)

    return k(uid, iid, a0, a1, it)


def _tc_body(g0_ref, g1_ref, v_ref, c0_ref, c1_ref, o_ref):
    l0 = g0_ref[...] * TEMP_INV
    l0 = l0 - jnp.max(l0, axis=1, keepdims=True)
    e0 = jnp.exp(l0)
    w0 = e0 / jnp.sum(e0, axis=1, keepdims=True)

    l1 = g1_ref[...] * TEMP_INV
    l1 = l1 - jnp.max(l1, axis=1, keepdims=True)
    e1 = jnp.exp(l1)
    w1 = e1 / jnp.sum(e1, axis=1, keepdims=True)

    u = (jnp.dot(w0, c0_ref[...], preferred_element_type=jnp.float32)
         + jnp.dot(w1, c1_ref[...], preferred_element_type=jnp.float32))

    o_ref[...] = jnp.sum(u * v_ref[...], axis=1, keepdims=True)


def _tc_compute(g0, g1, v, codebook0, codebook1):
    TB = 2048
    grid = (B // TB,)
    return pl.pallas_call(
        _tc_body,
        grid=grid,
        in_specs=[
            pl.BlockSpec((TB, C0), lambda i: (i, 0)),
            pl.BlockSpec((TB, C1), lambda i: (i, 0)),
            pl.BlockSpec((TB, D), lambda i: (i, 0)),
            pl.BlockSpec((C0, D), lambda i: (0, 0)),
            pl.BlockSpec((C1, D), lambda i: (0, 0)),
        ],
        out_specs=pl.BlockSpec((TB, 1), lambda i: (i, 0)),
        out_shape=jax.ShapeDtypeStruct((B, 1), jnp.float32),
    )(g0, g1, v, codebook0, codebook1)


def kernel(X, assign0, codebook0, assign1, codebook1, item_table):
    uid = X[:, 0]
    iid = X[:, 1]
    g0, g1, v = _sc_gather(uid, iid, assign0, assign1, item_table)
    return _tc_compute(g0, g1, v, codebook0, codebook1)


# speedup vs baseline: 2.1354x; 2.1354x over previous
import functools

import jax
import jax.numpy as jnp
from jax import lax
from jax.experimental import pallas as pl
from jax.experimental.pallas import tpu as pltpu
from jax.experimental.pallas import tpu_sc as plsc

TEMP_INV = 10.0

B = 16384
C0 = 64
C1 = 256
D = 32

NC, NS = 2, 16
NW = NC * NS
BPW = B // NW

H = 256
NH = BPW // H
VL = 16
NG = H // VL
A1_CHUNK = 64
A1_NCHUNK = BPW // A1_CHUNK


def _sc_gather(uid, iid, a0, a1, it):
    mesh = plsc.VectorSubcoreMesh(core_axis_name="c", subcore_axis_name="s")

    @functools.partial(
        pl.kernel,
        mesh=mesh,
        out_type=(
            jax.ShapeDtypeStruct((B, C0), jnp.float32),
            jax.ShapeDtypeStruct((B, C1), jnp.float32),
            jax.ShapeDtypeStruct((B, D), jnp.float32),
        ),
        scratch_types=[
            pltpu.VMEM((BPW,), jnp.int32),
            pltpu.VMEM((BPW,), jnp.int32),
            pltpu.VMEM((A1_NCHUNK, A1_CHUNK), jnp.int32),
            pltpu.VMEM((H, C0), jnp.float32),
            pltpu.VMEM((H, D), jnp.float32),
            pltpu.VMEM((A1_CHUNK, C1), jnp.float32),
            pltpu.VMEM((A1_CHUNK, C1), jnp.float32),
            pltpu.SemaphoreType.DMA,
            pltpu.SemaphoreType.DMA,
            pltpu.SemaphoreType.DMA,
            pltpu.SemaphoreType.DMA,
        ],
    )
    def k(uid_hbm, iid_hbm, a0_hbm, a1_hbm, it_hbm,
          g0_hbm, g1_hbm, v_hbm,
          uid_v, iid_v, uidx1_v, a0_buf, it_buf, a1_p, a1_q,
          s0, si, s1p, s1q):
        wid = lax.axis_index("s") * NC + lax.axis_index("c")
        base = wid * BPW
        pltpu.sync_copy(uid_hbm.at[pl.ds(base, BPW)], uid_v)
        pltpu.sync_copy(iid_hbm.at[pl.ds(base, BPW)], iid_v)
        for j in range(A1_NCHUNK):
            pltpu.sync_copy(
                uid_hbm.at[pl.ds(base + j * A1_CHUNK, A1_CHUNK)],
                uidx1_v.at[j])

        def fire1(kk):
            return pltpu.async_copy(
                a1_hbm.at[uidx1_v.at[kk]], (a1_p, a1_q)[kk % 2],
                (s1p, s1q)[kk % 2])

        cp1 = [None] * A1_NCHUNK
        cp1[0], cp1[1] = fire1(0), fire1(1)
        rnd = 0

        for h in range(NH):
            @pl.loop(0, NG)
            def _group(g):
                off = h * H + g * VL
                uvec = uid_v[pl.ds(off, VL)]
                ivec = iid_v[pl.ds(off, VL)]
                cps = []
                for t in range(VL):
                    cps.append(pltpu.async_copy(
                        a0_hbm.at[uvec[t]], a0_buf.at[g * VL + t], s0))
                    cps.append(pltpu.async_copy(
                        it_hbm.at[ivec[t]], it_buf.at[g * VL + t], si))
                for cp in cps:
                    cp.wait()
            pltpu.sync_copy(a0_buf, g0_hbm.at[pl.ds(base + h * H, H)])
            pltpu.sync_copy(it_buf, v_hbm.at[pl.ds(base + h * H, H)])
            for _ in range(A1_NCHUNK // NH):
                cp1[rnd].wait()
                pltpu.sync_copy(
                    (a1_p, a1_q)[rnd % 2],
                    g1_hbm.at[pl.ds(base + rnd * A1_CHUNK, A1_CHUNK)])
                if rnd + 2 < A1_NCHUNK:
                    cp1[rnd + 2] = fire1(rnd + 2)
                rnd += 1

    return k(uid, iid, a0, a1, it)


def _tc_body(g0_ref, g1_ref, v_ref, c0_ref, c1_ref, o_ref):
    l0 = g0_ref[...] * TEMP_INV
    l0 = l0 - jnp.max(l0, axis=1, keepdims=True)
    e0 = jnp.exp(l0)
    w0 = e0 / jnp.sum(e0, axis=1, keepdims=True)

    l1 = g1_ref[...] * TEMP_INV
    l1 = l1 - jnp.max(l1, axis=1, keepdims=True)
    e1 = jnp.exp(l1)
    w1 = e1 / jnp.sum(e1, axis=1, keepdims=True)

    u = (jnp.dot(w0, c0_ref[...], preferred_element_type=jnp.float32)
         + jnp.dot(w1, c1_ref[...], preferred_element_type=jnp.float32))

    o_ref[...] = jnp.sum(u * v_ref[...], axis=1, keepdims=True)


def _tc_compute(g0, g1, v, codebook0, codebook1):
    TB = 2048
    grid = (B // TB,)
    return pl.pallas_call(
        _tc_body,
        grid=grid,
        in_specs=[
            pl.BlockSpec((TB, C0), lambda i: (i, 0)),
            pl.BlockSpec((TB, C1), lambda i: (i, 0)),
            pl.BlockSpec((TB, D), lambda i: (i, 0)),
            pl.BlockSpec((C0, D), lambda i: (0, 0)),
            pl.BlockSpec((C1, D), lambda i: (0, 0)),
        ],
        out_specs=pl.BlockSpec((TB, 1), lambda i: (i, 0)),
        out_shape=jax.ShapeDtypeStruct((B, 1), jnp.float32),
    )(g0, g1, v, codebook0, codebook1)


def kernel(X, assign0, codebook0, assign1, codebook1, item_table):
    uid = X[:, 0]
    iid = X[:, 1]
    g0, g1, v = _sc_gather(uid, iid, assign0, assign1, item_table)
    return _tc_compute(g0, g1, v, codebook0, codebook1)


# speedup vs baseline: 2.1968x; 1.0287x over previous
import functools

import jax
import jax.numpy as jnp
from jax import lax
from jax.experimental import pallas as pl
from jax.experimental.pallas import tpu as pltpu
from jax.experimental.pallas import tpu_sc as plsc

TEMP_INV = 10.0

B = 16384
C0 = 64
C1 = 256
D = 32

NC, NS = 2, 16
NW = NC * NS
BPW = B // NW

H = 256
NH = BPW // H
VL = 16
GV = 4
GR = VL * GV
NG = H // GR
NSEM = 4
A1_CHUNK = 64
A1_NCHUNK = BPW // A1_CHUNK


def _sc_gather(uid, iid, a0, a1, it):
    mesh = plsc.VectorSubcoreMesh(core_axis_name="c", subcore_axis_name="s")

    @functools.partial(
        pl.kernel,
        mesh=mesh,
        out_type=(
            jax.ShapeDtypeStruct((B, C0), jnp.float32),
            jax.ShapeDtypeStruct((B, C1), jnp.float32),
            jax.ShapeDtypeStruct((B, D), jnp.float32),
        ),
        scratch_types=[
            pltpu.VMEM((BPW,), jnp.int32),
            pltpu.VMEM((BPW,), jnp.int32),
            pltpu.VMEM((A1_NCHUNK, A1_CHUNK), jnp.int32),
            pltpu.VMEM((H, C0), jnp.float32),
            pltpu.VMEM((H, D), jnp.float32),
            pltpu.VMEM((A1_CHUNK, C1), jnp.float32),
            pltpu.VMEM((A1_CHUNK, C1), jnp.float32),
            [pltpu.SemaphoreType.DMA] * NSEM,
            [pltpu.SemaphoreType.DMA] * NSEM,
            pltpu.SemaphoreType.DMA,
            pltpu.SemaphoreType.DMA,
        ],
    )
    def k(uid_hbm, iid_hbm, a0_hbm, a1_hbm, it_hbm,
          g0_hbm, g1_hbm, v_hbm,
          uid_v, iid_v, uidx1_v, a0_buf, it_buf, a1_p, a1_q,
          s0, si, s1p, s1q):
        wid = lax.axis_index("s") * NC + lax.axis_index("c")
        base = wid * BPW
        pltpu.sync_copy(uid_hbm.at[pl.ds(base, BPW)], uid_v)
        pltpu.sync_copy(iid_hbm.at[pl.ds(base, BPW)], iid_v)
        for j in range(A1_NCHUNK):
            pltpu.sync_copy(
                uid_hbm.at[pl.ds(base + j * A1_CHUNK, A1_CHUNK)],
                uidx1_v.at[j])

        def fire1(kk):
            return pltpu.async_copy(
                a1_hbm.at[uidx1_v.at[kk]], (a1_p, a1_q)[kk % 2],
                (s1p, s1q)[kk % 2])

        cp1 = [None] * A1_NCHUNK
        cp1[0], cp1[1] = fire1(0), fire1(1)
        rnd = 0

        for h in range(NH):
            @pl.loop(0, NG)
            def _group(g):
                cps = []
                for q in range(GV):
                    off = h * H + g * GR + q * VL
                    uvec = uid_v[pl.ds(off, VL)]
                    ivec = iid_v[pl.ds(off, VL)]
                    for t in range(VL):
                        r = g * GR + q * VL + t
                        cps.append(pltpu.async_copy(
                            a0_hbm.at[uvec[t]], a0_buf.at[r],
                            s0[(q * VL + t) % NSEM]))
                        cps.append(pltpu.async_copy(
                            it_hbm.at[ivec[t]], it_buf.at[r],
                            si[(q * VL + t) % NSEM]))
                for cp in cps:
                    cp.wait()
            pltpu.sync_copy(a0_buf, g0_hbm.at[pl.ds(base + h * H, H)])
            pltpu.sync_copy(it_buf, v_hbm.at[pl.ds(base + h * H, H)])
            for _ in range(A1_NCHUNK // NH):
                cp1[rnd].wait()
                pltpu.sync_copy(
                    (a1_p, a1_q)[rnd % 2],
                    g1_hbm.at[pl.ds(base + rnd * A1_CHUNK, A1_CHUNK)])
                if rnd + 2 < A1_NCHUNK:
                    cp1[rnd + 2] = fire1(rnd + 2)
                rnd += 1

    return k(uid, iid, a0, a1, it)


def _tc_body(g0_ref, g1_ref, v_ref, c0_ref, c1_ref, o_ref):
    l0 = g0_ref[...] * TEMP_INV
    l0 = l0 - jnp.max(l0, axis=1, keepdims=True)
    e0 = jnp.exp(l0)
    w0 = e0 / jnp.sum(e0, axis=1, keepdims=True)

    l1 = g1_ref[...] * TEMP_INV
    l1 = l1 - jnp.max(l1, axis=1, keepdims=True)
    e1 = jnp.exp(l1)
    w1 = e1 / jnp.sum(e1, axis=1, keepdims=True)

    u = (jnp.dot(w0, c0_ref[...], preferred_element_type=jnp.float32)
         + jnp.dot(w1, c1_ref[...], preferred_element_type=jnp.float32))

    o_ref[...] = jnp.sum(u * v_ref[...], axis=1, keepdims=True)


def _tc_compute(g0, g1, v, codebook0, codebook1):
    TB = 2048
    grid = (B // TB,)
    return pl.pallas_call(
        _tc_body,
        grid=grid,
        in_specs=[
            pl.BlockSpec((TB, C0), lambda i: (i, 0)),
            pl.BlockSpec((TB, C1), lambda i: (i, 0)),
            pl.BlockSpec((TB, D), lambda i: (i, 0)),
            pl.BlockSpec((C0, D), lambda i: (0, 0)),
            pl.BlockSpec((C1, D), lambda i: (0, 0)),
        ],
        out_specs=pl.BlockSpec((TB, 1), lambda i: (i, 0)),
        out_shape=jax.ShapeDtypeStruct((B, 1), jnp.float32),
    )(g0, g1, v, codebook0, codebook1)


def kernel(X, assign0, codebook0, assign1, codebook1, item_table):
    uid = X[:, 0]
    iid = X[:, 1]
    g0, g1, v = _sc_gather(uid, iid, assign0, assign1, item_table)
    return _tc_compute(g0, g1, v, codebook0, codebook1)
